# baseline (speedup 1.0000x reference)
import functools

import jax
import jax.numpy as jnp
from jax import lax
from jax.experimental import layout as jex_layout
from jax.experimental import pallas as pl
from jax.experimental.pallas import tpu as pltpu
from jax.experimental.pallas import tpu_sc as plsc

B = 4096
H = 200
E = 64
OUTD = 128
CA = 128
CB = H - CA
VOCAB = 1000000


def _make_sc_pool():
    info = plsc.get_sparse_core_info()
    nc, ns = info.num_cores, info.num_subcores
    nw = nc * ns
    bpw = B // nw
    mesh = plsc.VectorSubcoreMesh(core_axis_name="c", subcore_axis_name="s")

    @functools.partial(
        pl.kernel,
        out_type=jax.ShapeDtypeStruct((B, E), jnp.float32),
        mesh=mesh,
        scratch_types=[
            pltpu.VMEM((bpw, H), jnp.int32),
            pltpu.VMEM((2, CA, E), jnp.float32),
            pltpu.VMEM((2, CB, E), jnp.float32),
            pltpu.VMEM((bpw, E), jnp.float32),
            pltpu.SemaphoreType.DMA,
            pltpu.SemaphoreType.DMA,
        ],
        compiler_params=pltpu.CompilerParams(use_tc_tiling_on_sc=False),
    )
    def pool(x_hbm, table_hbm, out_hbm,
             idx_v, rowsa_v, rowsb_v, pooled_v, sem0, sem1):
        sems = (sem0, sem1)
        wid = lax.axis_index("s") * nc + lax.axis_index("c")
        base = wid * bpw
        pltpu.sync_copy(x_hbm.at[pl.ds(base, bpw)], idx_v)

        def start(b, par):
            pltpu.async_copy(
                table_hbm.at[idx_v.at[b, pl.ds(0, CA)]], rowsa_v.at[par],
                sems[par],
            )
            pltpu.async_copy(
                table_hbm.at[idx_v.at[b, pl.ds(CA, CB)]], rowsb_v.at[par],
                sems[par],
            )

        def wait(b, par):
            pltpu.make_async_copy(
                table_hbm.at[idx_v.at[b, pl.ds(0, CA)]], rowsa_v.at[par],
                sems[par],
            ).wait()
            pltpu.make_async_copy(
                table_hbm.at[idx_v.at[b, pl.ds(CA, CB)]], rowsb_v.at[par],
                sems[par],
            ).wait()

        start(0, 0)

        def outer(g, _):
            for par in range(2):
                b = 2 * g + par

                @pl.when(b + 1 < bpw)
                def _():
                    start(b + 1, (par + 1) % 2)

                wait(b, par)

                def add_rows(accs, rows_ref, slot0, ks):
                    accs = list(accs)
                    for k in ks:
                        for j in range(E // 16):
                            i = j * 2 + (k % 2)
                            accs[i] = accs[i] + rows_ref[
                                par, slot0 + k - ks[0], pl.ds(j * 16, 16)
                            ]
                    return tuple(accs)

                zero = jnp.zeros((16,), jnp.float32)
                accs = (zero,) * (2 * (E // 16))
                accs = lax.fori_loop(
                    0, CA // 16,
                    lambda g16, a: add_rows(a, rowsa_v, g16 * 16, range(16)),
                    accs,
                )
                accs = lax.fori_loop(
                    0, CB // 16,
                    lambda g16, a: add_rows(a, rowsb_v, g16 * 16, range(16)),
                    accs,
                )
                accs = add_rows(accs, rowsb_v, CB - 8, range(8))
                for j in range(E // 16):
                    s = accs[j * 2] + accs[j * 2 + 1]
                    pooled_v[b, pl.ds(j * 16, 16)] = s
            return 0

        lax.fori_loop(0, bpw // 2, outer, 0)
        pltpu.sync_copy(pooled_v, out_hbm.at[pl.ds(base, bpw)])

    return pool


def _tc_proj(pooled_sum, W, b):
    blk = 512

    def body(p_ref, w_ref, b_ref, o_ref):
        o_ref[...] = (
            jnp.dot(
                p_ref[...] * (1.0 / H), w_ref[...],
                preferred_element_type=jnp.float32,
            )
            + b_ref[...]
        )

    return pl.pallas_call(
        body,
        grid=(B // blk,),
        in_specs=[
            pl.BlockSpec((blk, E), lambda i: (i, 0)),
            pl.BlockSpec((E, OUTD), lambda i: (0, 0)),
            pl.BlockSpec((1, OUTD), lambda i: (0, 0)),
        ],
        out_specs=pl.BlockSpec((blk, OUTD), lambda i: (i, 0)),
        out_shape=jax.ShapeDtypeStruct((B, OUTD), jnp.float32),
    )(pooled_sum, W, b.reshape(1, OUTD))


def kernel(x, table, W, b):
    x2 = x.astype(jnp.int32) << 1
    nslab = 8
    sr = VOCAB // nslab
    padded = [
        jnp.pad(lax.slice(table, (k * sr, 0), ((k + 1) * sr, E)),
                ((0, 0), (0, E)))
        for k in range(nslab)
    ]
    table2 = jnp.concatenate(padded, axis=0).reshape(2 * VOCAB, E)
    pooled_sum = _make_sc_pool()(x2, table2)
    return _tc_proj(pooled_sum, W, b)


# speedup vs baseline: 2.8353x; 2.8353x over previous
import functools

import jax
import jax.numpy as jnp
from jax import lax
from jax.experimental import pallas as pl
from jax.experimental.pallas import tpu as pltpu
from jax.experimental.pallas import tpu_sc as plsc

B = 4096
H = 200
E = 64
OUTD = 128
CA = 128
CB = H - CA
VOCAB = 1000000


def _make_sc_pool():
    info = plsc.get_sparse_core_info()
    nc, ns = info.num_cores, info.num_subcores
    nw = nc * ns
    bpw = B // nw
    mesh = plsc.VectorSubcoreMesh(core_axis_name="c", subcore_axis_name="s")

    @functools.partial(
        pl.kernel,
        out_type=jax.ShapeDtypeStruct((B, E), jnp.float32),
        mesh=mesh,
        scratch_types=[
            pltpu.VMEM((bpw, H), jnp.int32),
            pltpu.VMEM((2, CA, E), jnp.float32),
            pltpu.VMEM((2, CB, E), jnp.float32),
            pltpu.VMEM((bpw, E), jnp.float32),
            pltpu.SemaphoreType.DMA,
            pltpu.SemaphoreType.DMA,
        ],
        compiler_params=pltpu.CompilerParams(use_tc_tiling_on_sc=False),
    )
    def pool(x_hbm, table_hbm, out_hbm,
             idx_v, rowsa_v, rowsb_v, pooled_v, sem0, sem1):
        sems = (sem0, sem1)
        wid = lax.axis_index("s") * nc + lax.axis_index("c")
        base = wid * bpw
        pltpu.sync_copy(x_hbm.at[pl.ds(base, bpw)], idx_v)

        def start(b, par):
            pltpu.async_copy(
                table_hbm.at[idx_v.at[b, pl.ds(0, CA)]], rowsa_v.at[par],
                sems[par],
            )
            pltpu.async_copy(
                table_hbm.at[idx_v.at[b, pl.ds(CA, CB)]], rowsb_v.at[par],
                sems[par],
            )

        def wait(b, par):
            pltpu.make_async_copy(
                table_hbm.at[idx_v.at[b, pl.ds(0, CA)]], rowsa_v.at[par],
                sems[par],
            ).wait()
            pltpu.make_async_copy(
                table_hbm.at[idx_v.at[b, pl.ds(CA, CB)]], rowsb_v.at[par],
                sems[par],
            ).wait()

        start(0, 0)

        def outer(g, _):
            for par in range(2):
                b = 2 * g + par

                @pl.when(b + 1 < bpw)
                def _():
                    start(b + 1, (par + 1) % 2)

                wait(b, par)

                def add_rows(accs, rows_ref, slot0, ks):
                    accs = list(accs)
                    for k in ks:
                        for j in range(E // 16):
                            i = j * 2 + (k % 2)
                            accs[i] = accs[i] + rows_ref[
                                par, slot0 + k - ks[0], pl.ds(j * 16, 16)
                            ]
                    return tuple(accs)

                zero = jnp.zeros((16,), jnp.float32)
                accs = (zero,) * (2 * (E // 16))
                accs = lax.fori_loop(
                    0, CA // 16,
                    lambda g16, a: add_rows(a, rowsa_v, g16 * 16, range(16)),
                    accs,
                )
                accs = lax.fori_loop(
                    0, CB // 16,
                    lambda g16, a: add_rows(a, rowsb_v, g16 * 16, range(16)),
                    accs,
                )
                accs = add_rows(accs, rowsb_v, CB - 8, range(8))
                for j in range(E // 16):
                    s = accs[j * 2] + accs[j * 2 + 1]
                    pooled_v[b, pl.ds(j * 16, 16)] = s
            return 0

        lax.fori_loop(0, bpw // 2, outer, 0)
        pltpu.sync_copy(pooled_v, out_hbm.at[pl.ds(base, bpw)])

    return pool


def _tc_pad(table):
    blkr = 4096

    def body(t_ref, o_ref):
        o_ref[...] = jnp.concatenate(
            [t_ref[...], jnp.zeros((blkr, E), jnp.float32)], axis=1
        )

    return pl.pallas_call(
        body,
        grid=(pl.cdiv(VOCAB, blkr),),
        in_specs=[pl.BlockSpec((blkr, E), lambda i: (i, 0))],
        out_specs=pl.BlockSpec((blkr, 2 * E), lambda i: (i, 0)),
        out_shape=jax.ShapeDtypeStruct((VOCAB, 2 * E), jnp.float32),
    )(table)


def _tc_proj(pooled_sum, W, b):
    blk = 512

    def body(p_ref, w_ref, b_ref, o_ref):
        o_ref[...] = (
            jnp.dot(
                p_ref[...] * (1.0 / H), w_ref[...],
                preferred_element_type=jnp.float32,
            )
            + b_ref[...]
        )

    return pl.pallas_call(
        body,
        grid=(B // blk,),
        in_specs=[
            pl.BlockSpec((blk, E), lambda i: (i, 0)),
            pl.BlockSpec((E, OUTD), lambda i: (0, 0)),
            pl.BlockSpec((1, OUTD), lambda i: (0, 0)),
        ],
        out_specs=pl.BlockSpec((blk, OUTD), lambda i: (i, 0)),
        out_shape=jax.ShapeDtypeStruct((B, OUTD), jnp.float32),
    )(pooled_sum, W, b.reshape(1, OUTD))


def kernel(x, table, W, b):
    x2 = x.astype(jnp.int32) << 1
    table2 = _tc_pad(table).reshape(2 * VOCAB, E)
    pooled_sum = _make_sc_pool()(x2, table2)
    return _tc_proj(pooled_sum, W, b)


# speedup vs baseline: 3.5167x; 1.2403x over previous
import functools

import jax
import jax.numpy as jnp
from jax import lax
from jax.experimental import pallas as pl
from jax.experimental.pallas import tpu as pltpu
from jax.experimental.pallas import tpu_sc as plsc

B = 4096
H = 200
E = 64
OUTD = 128
CA = 128
CB = H - CA
VOCAB = 1000000


def _make_sc_pool():
    info = plsc.get_sparse_core_info()
    nc, ns = info.num_cores, info.num_subcores
    nw = nc * ns
    bpw = B // nw
    mesh = plsc.VectorSubcoreMesh(core_axis_name="c", subcore_axis_name="s")

    @functools.partial(
        pl.kernel,
        out_type=jax.ShapeDtypeStruct((B, E), jnp.float32),
        mesh=mesh,
        scratch_types=[
            pltpu.VMEM((bpw, H), jnp.int32),
            pltpu.VMEM((2, CA, E), jnp.float32),
            pltpu.VMEM((2, CB, E), jnp.float32),
            pltpu.VMEM((bpw, E), jnp.float32),
            pltpu.SemaphoreType.DMA,
            pltpu.SemaphoreType.DMA,
        ],
        compiler_params=pltpu.CompilerParams(use_tc_tiling_on_sc=False),
    )
    def pool(x_hbm, table_hbm, out_hbm,
             idx_v, rowsa_v, rowsb_v, pooled_v, sem0, sem1):
        sems = (sem0, sem1)
        wid = lax.axis_index("s") * nc + lax.axis_index("c")
        base = wid * bpw
        pltpu.sync_copy(x_hbm.at[pl.ds(base, bpw)], idx_v)

        def start(b, par):
            pltpu.async_copy(
                table_hbm.at[idx_v.at[b, pl.ds(0, CA)]], rowsa_v.at[par],
                sems[par],
            )
            pltpu.async_copy(
                table_hbm.at[idx_v.at[b, pl.ds(CA, CB)]], rowsb_v.at[par],
                sems[par],
            )

        def wait(b, par):
            pltpu.make_async_copy(
                table_hbm.at[idx_v.at[b, pl.ds(0, CA)]], rowsa_v.at[par],
                sems[par],
            ).wait()
            pltpu.make_async_copy(
                table_hbm.at[idx_v.at[b, pl.ds(CA, CB)]], rowsb_v.at[par],
                sems[par],
            ).wait()

        start(0, 0)

        def outer(g, _):
            for par in range(2):
                b = 2 * g + par

                @pl.when(b + 1 < bpw)
                def _():
                    start(b + 1, (par + 1) % 2)

                wait(b, par)

                def add_rows(accs, rows_ref, slot0, ks):
                    accs = list(accs)
                    for k in ks:
                        for j in range(E // 16):
                            i = j * 2 + (k % 2)
                            accs[i] = accs[i] + rows_ref[
                                par, slot0 + k - ks[0], pl.ds(j * 16, 16)
                            ]
                    return tuple(accs)

                zero = jnp.zeros((16,), jnp.float32)
                accs = (zero,) * (2 * (E // 16))
                accs = lax.fori_loop(
                    0, CA // 16,
                    lambda g16, a: add_rows(a, rowsa_v, g16 * 16, range(16)),
                    accs,
                )
                accs = lax.fori_loop(
                    0, CB // 16,
                    lambda g16, a: add_rows(a, rowsb_v, g16 * 16, range(16)),
                    accs,
                )
                accs = add_rows(accs, rowsb_v, CB - 8, range(8))
                for j in range(E // 16):
                    s = accs[j * 2] + accs[j * 2 + 1]
                    pooled_v[b, pl.ds(j * 16, 16)] = s
            return 0

        lax.fori_loop(0, bpw // 2, outer, 0)
        pltpu.sync_copy(pooled_v, out_hbm.at[pl.ds(base, bpw)])

    return pool


def _tc_proj(pooled_sum, W, b):
    blk = 512

    def body(p_ref, w_ref, b_ref, o_ref):
        o_ref[...] = (
            jnp.dot(
                p_ref[...] * (1.0 / H), w_ref[...],
                preferred_element_type=jnp.float32,
            )
            + b_ref[...]
        )

    return pl.pallas_call(
        body,
        grid=(B // blk,),
        in_specs=[
            pl.BlockSpec((blk, E), lambda i: (i, 0)),
            pl.BlockSpec((E, OUTD), lambda i: (0, 0)),
            pl.BlockSpec((1, OUTD), lambda i: (0, 0)),
        ],
        out_specs=pl.BlockSpec((blk, OUTD), lambda i: (i, 0)),
        out_shape=jax.ShapeDtypeStruct((B, OUTD), jnp.float32),
    )(pooled_sum, W, b.reshape(1, OUTD))


def kernel(x, table, W, b):
    x2 = x.astype(jnp.int32) << 1
    table2 = jnp.pad(table, ((0, 0), (0, E))).reshape(2 * VOCAB, E)
    pooled_sum = _make_sc_pool()(x2, table2)
    return _tc_proj(pooled_sum, W, b)


# speedup vs baseline: 3.6846x; 1.0477x over previous
import functools

import jax
import jax.numpy as jnp
from jax import lax
from jax.experimental import pallas as pl
from jax.experimental.pallas import tpu as pltpu
from jax.experimental.pallas import tpu_sc as plsc

B = 4096
H = 200
E = 64
OUTD = 128
CA = 128
CB = H - CA
VOCAB = 1000000


def _make_sc_pool():
    info = plsc.get_sparse_core_info()
    nc, ns = info.num_cores, info.num_subcores
    nw = nc * ns
    bpw = B // nw
    mesh = plsc.VectorSubcoreMesh(core_axis_name="c", subcore_axis_name="s")

    @functools.partial(
        pl.kernel,
        out_type=jax.ShapeDtypeStruct((B, E), jnp.float32),
        mesh=mesh,
        scratch_types=[
            pltpu.VMEM((bpw, H), jnp.int32),
            pltpu.VMEM((4, CA, E), jnp.float32),
            pltpu.VMEM((4, CB, E), jnp.float32),
            pltpu.VMEM((bpw, E), jnp.float32),
            pltpu.SemaphoreType.DMA,
            pltpu.SemaphoreType.DMA,
            pltpu.SemaphoreType.DMA,
            pltpu.SemaphoreType.DMA,
        ],
        compiler_params=pltpu.CompilerParams(use_tc_tiling_on_sc=False),
    )
    def pool(x_hbm, table_hbm, out_hbm,
             idx_v, rowsa_v, rowsb_v, pooled_v, sem0, sem1, sem2, sem3):
        sems = (sem0, sem1, sem2, sem3)
        wid = lax.axis_index("s") * nc + lax.axis_index("c")
        base = wid * bpw
        pltpu.sync_copy(x_hbm.at[pl.ds(base, bpw)], idx_v)

        def start(b, par):
            pltpu.async_copy(
                table_hbm.at[idx_v.at[b, pl.ds(0, CA)]], rowsa_v.at[par],
                sems[par],
            )
            pltpu.async_copy(
                table_hbm.at[idx_v.at[b, pl.ds(CA, CB)]], rowsb_v.at[par],
                sems[par],
            )

        def wait(b, par):
            pltpu.make_async_copy(
                table_hbm.at[idx_v.at[b, pl.ds(0, CA)]], rowsa_v.at[par],
                sems[par],
            ).wait()
            pltpu.make_async_copy(
                table_hbm.at[idx_v.at[b, pl.ds(CA, CB)]], rowsb_v.at[par],
                sems[par],
            ).wait()

        start(0, 0)
        start(1, 1)

        def outer(g, _):
            for par in range(4):
                b = 4 * g + par

                @pl.when(b + 2 < bpw)
                def _():
                    start(b + 2, (par + 2) % 4)

                wait(b, par)

                def add_rows(accs, rows_ref, slot0, ks):
                    accs = list(accs)
                    for k in ks:
                        for j in range(E // 16):
                            i = j * 2 + (k % 2)
                            accs[i] = accs[i] + rows_ref[
                                par, slot0 + k - ks[0], pl.ds(j * 16, 16)
                            ]
                    return tuple(accs)

                zero = jnp.zeros((16,), jnp.float32)
                accs = (zero,) * (2 * (E // 16))
                accs = lax.fori_loop(
                    0, CA // 16,
                    lambda g16, a: add_rows(a, rowsa_v, g16 * 16, range(16)),
                    accs,
                )
                accs = lax.fori_loop(
                    0, CB // 16,
                    lambda g16, a: add_rows(a, rowsb_v, g16 * 16, range(16)),
                    accs,
                )
                accs = add_rows(accs, rowsb_v, CB - 8, range(8))
                for j in range(E // 16):
                    s = accs[j * 2] + accs[j * 2 + 1]
                    pooled_v[b, pl.ds(j * 16, 16)] = s
            return 0

        lax.fori_loop(0, bpw // 4, outer, 0)
        pltpu.sync_copy(pooled_v, out_hbm.at[pl.ds(base, bpw)])

    return pool


def _tc_proj(pooled_sum, W, b):
    blk = 512

    def body(p_ref, w_ref, b_ref, o_ref):
        o_ref[...] = (
            jnp.dot(
                p_ref[...] * (1.0 / H), w_ref[...],
                preferred_element_type=jnp.float32,
            )
            + b_ref[...]
        )

    return pl.pallas_call(
        body,
        grid=(B // blk,),
        in_specs=[
            pl.BlockSpec((blk, E), lambda i: (i, 0)),
            pl.BlockSpec((E, OUTD), lambda i: (0, 0)),
            pl.BlockSpec((1, OUTD), lambda i: (0, 0)),
        ],
        out_specs=pl.BlockSpec((blk, OUTD), lambda i: (i, 0)),
        out_shape=jax.ShapeDtypeStruct((B, OUTD), jnp.float32),
    )(pooled_sum, W, b.reshape(1, OUTD))


def kernel(x, table, W, b):
    x2 = x.astype(jnp.int32) << 1
    table2 = jnp.pad(table, ((0, 0), (0, E))).reshape(2 * VOCAB, E)
    pooled_sum = _make_sc_pool()(x2, table2)
    return _tc_proj(pooled_sum, W, b)


# speedup vs baseline: 3.7399x; 1.0150x over previous
import functools

import jax
import jax.numpy as jnp
from jax import lax
from jax.experimental import pallas as pl
from jax.experimental.pallas import tpu as pltpu
from jax.experimental.pallas import tpu_sc as plsc

B = 4096
H = 200
E = 64
OUTD = 128
CA = 128
CB = H - CA
VOCAB = 1000000


def _make_sc_pool():
    info = plsc.get_sparse_core_info()
    nc, ns = info.num_cores, info.num_subcores
    nw = nc * ns
    bpw = B // nw
    mesh = plsc.VectorSubcoreMesh(core_axis_name="c", subcore_axis_name="s")

    @functools.partial(
        pl.kernel,
        out_type=jax.ShapeDtypeStruct((B, E), jnp.float32),
        mesh=mesh,
        scratch_types=[
            pltpu.VMEM((bpw, H), jnp.int32),
            pltpu.VMEM((4, CA, E), jnp.float32),
            pltpu.VMEM((4, CB, E), jnp.float32),
            pltpu.VMEM((bpw, E), jnp.float32),
            pltpu.SemaphoreType.DMA,
            pltpu.SemaphoreType.DMA,
            pltpu.SemaphoreType.DMA,
            pltpu.SemaphoreType.DMA,
        ],
        compiler_params=pltpu.CompilerParams(use_tc_tiling_on_sc=False),
    )
    def pool(x_hbm, table_hbm, out_hbm,
             idx_v, rowsa_v, rowsb_v, pooled_v, sem0, sem1, sem2, sem3):
        sems = (sem0, sem1, sem2, sem3)
        wid = lax.axis_index("s") * nc + lax.axis_index("c")
        base = wid * bpw
        pltpu.sync_copy(x_hbm.at[pl.ds(base, bpw)], idx_v)

        def start(b, par):
            pltpu.async_copy(
                table_hbm.at[idx_v.at[b, pl.ds(0, CA)]], rowsa_v.at[par],
                sems[par],
            )
            pltpu.async_copy(
                table_hbm.at[idx_v.at[b, pl.ds(CA, CB)]], rowsb_v.at[par],
                sems[par],
            )

        def wait(b, par):
            pltpu.make_async_copy(
                table_hbm.at[idx_v.at[b, pl.ds(0, CA)]], rowsa_v.at[par],
                sems[par],
            ).wait()
            pltpu.make_async_copy(
                table_hbm.at[idx_v.at[b, pl.ds(CA, CB)]], rowsb_v.at[par],
                sems[par],
            ).wait()

        start(0, 0)
        start(1, 1)
        start(2, 2)

        def outer(g, _):
            for par in range(4):
                b = 4 * g + par

                @pl.when(b + 3 < bpw)
                def _():
                    start(b + 3, (par + 3) % 4)

                wait(b, par)

                def add_rows(accs, rows_ref, slot0, ks):
                    accs = list(accs)
                    for k in ks:
                        for j in range(E // 16):
                            i = j * 2 + (k % 2)
                            accs[i] = accs[i] + rows_ref[
                                par, slot0 + k - ks[0], pl.ds(j * 16, 16)
                            ]
                    return tuple(accs)

                zero = jnp.zeros((16,), jnp.float32)
                accs = (zero,) * (2 * (E // 16))
                accs = lax.fori_loop(
                    0, CA // 16,
                    lambda g16, a: add_rows(a, rowsa_v, g16 * 16, range(16)),
                    accs,
                )
                accs = lax.fori_loop(
                    0, CB // 16,
                    lambda g16, a: add_rows(a, rowsb_v, g16 * 16, range(16)),
                    accs,
                )
                accs = add_rows(accs, rowsb_v, CB - 8, range(8))
                for j in range(E // 16):
                    s = accs[j * 2] + accs[j * 2 + 1]
                    pooled_v[b, pl.ds(j * 16, 16)] = s
            return 0

        lax.fori_loop(0, bpw // 4, outer, 0)
        pltpu.sync_copy(pooled_v, out_hbm.at[pl.ds(base, bpw)])

    return pool


def _tc_proj(pooled_sum, W, b):
    blk = 512

    def body(p_ref, w_ref, b_ref, o_ref):
        o_ref[...] = (
            jnp.dot(
                p_ref[...] * (1.0 / H), w_ref[...],
                preferred_element_type=jnp.float32,
            )
            + b_ref[...]
        )

    return pl.pallas_call(
        body,
        grid=(B // blk,),
        in_specs=[
            pl.BlockSpec((blk, E), lambda i: (i, 0)),
            pl.BlockSpec((E, OUTD), lambda i: (0, 0)),
            pl.BlockSpec((1, OUTD), lambda i: (0, 0)),
        ],
        out_specs=pl.BlockSpec((blk, OUTD), lambda i: (i, 0)),
        out_shape=jax.ShapeDtypeStruct((B, OUTD), jnp.float32),
    )(pooled_sum, W, b.reshape(1, OUTD))


def kernel(x, table, W, b):
    x2 = x.astype(jnp.int32) << 1
    table2 = jnp.pad(table, ((0, 0), (0, E))).reshape(2 * VOCAB, E)
    pooled_sum = _make_sc_pool()(x2, table2)
    return _tc_proj(pooled_sum, W, b)
